# baseline (speedup 1.0000x reference)
import functools

import jax
import jax.numpy as jnp
from jax import lax
from jax.experimental import pallas as pl
from jax.experimental.pallas import tpu as pltpu
from jax.experimental.pallas import tpu_sc as plsc

N_NODES = 10000
N_EDGES = 320000
D_FEAT = 128
D_HALF = 64
D_SCALAR = 16
D_HIDDEN = 64

NC = 2
NS = 16
NW = NC * NS
C = 32
NCHUNK = N_EDGES // C
_CQ, _CR = divmod(NCHUNK, NW)
PMAX = (_CQ + 1 + 1) // 2
NG = C // 16


def _pack_pairs(w):
    lo = lax.bitcast_convert_type(w[:, :D_HALF].astype(jnp.bfloat16), jnp.uint16)
    hi = lax.bitcast_convert_type(w[:, D_HALF:].astype(jnp.bfloat16), jnp.uint16)
    word = lo.astype(jnp.uint32) | (hi.astype(jnp.uint32) << 16)
    return lax.bitcast_convert_type(word, jnp.float32)


def _g_body(nf_ref, wdot_ref, tab_ref):
    nf = nf_ref[...]
    g = jnp.dot(nf, wdot_ref[...], preferred_element_type=jnp.float32)
    tab_ref[:, :D_HALF] = _pack_pairs(nf)
    tab_ref[:, D_HALF:] = _pack_pairs(g)


def _node_prep(node_f, W_dot):
    return pl.pallas_call(
        _g_body,
        out_shape=jax.ShapeDtypeStruct((N_NODES, D_FEAT), jnp.float32),
    )(node_f, W_dot)


_BE = 8000


def _mlp_body(esc_ref, cut_ref, wk1_ref, wk2_ref, wv1_ref, wv2_ref,
              wkv_ref, sq_ref):
    esc = esc_ref[...]
    hk = jnp.maximum(jnp.dot(esc, wk1_ref[...],
                             preferred_element_type=jnp.float32), 0.0)
    wk = jnp.dot(hk, wk2_ref[...],
                 preferred_element_type=jnp.float32) * (1.0 / 32.0)
    hv = jnp.maximum(jnp.dot(esc, wv1_ref[...],
                             preferred_element_type=jnp.float32), 0.0)
    wv = jnp.dot(hv, wv2_ref[...],
                 preferred_element_type=jnp.float32) * (1.0 / 32.0)
    wkv_ref[:, :D_HALF] = _pack_pairs(wk)
    wkv_ref[:, D_HALF:] = _pack_pairs(wv)
    sq_ref[...] = jnp.sqrt(cut_ref[...])


def _compute_mlps(esc, cutoff, W_k1, W_k2, W_v1, W_v2):
    nb = N_EDGES // _BE
    cut3 = cutoff.reshape(nb, 1, _BE)
    wkv, sq3 = pl.pallas_call(
        _mlp_body,
        grid=(nb,),
        in_specs=[
            pl.BlockSpec((_BE, D_SCALAR), lambda i: (i, 0)),
            pl.BlockSpec((1, 1, _BE), lambda i: (i, 0, 0)),
            pl.BlockSpec((D_SCALAR, D_HIDDEN), lambda i: (0, 0)),
            pl.BlockSpec((D_HIDDEN, D_FEAT), lambda i: (0, 0)),
            pl.BlockSpec((D_SCALAR, D_HIDDEN), lambda i: (0, 0)),
            pl.BlockSpec((D_HIDDEN, D_FEAT), lambda i: (0, 0)),
        ],
        out_specs=[
            pl.BlockSpec((_BE, D_FEAT), lambda i: (i, 0)),
            pl.BlockSpec((1, 1, _BE), lambda i: (i, 0, 0)),
        ],
        out_shape=[
            jax.ShapeDtypeStruct((N_EDGES, D_FEAT), jnp.float32),
            jax.ShapeDtypeStruct((nb, 1, _BE), jnp.float32),
        ],
    )(esc, cut3, W_k1, W_k2, W_v1, W_v2)
    return wkv, sq3.reshape(N_EDGES)


def _sc_body(tab_hbm, wkv_hbm, sq_hbm, ea_hbm, src_hbm, dst_hbm,
             zac_hbm, zz_hbm,
             acc_out, z_out,
             srcv0, dstv0, srcv1, dstv1,
             xs0, gd0, wkv0, sqb0, eab0,
             xs1, gd1, wkv1, sqb1, eab1,
             tb, eb, dotbuf,
             acc_sh, z_sh, semidx0, semidx1, semld0, semld1):
    cid = lax.axis_index("c")
    sid = lax.axis_index("s")
    wid = sid * NC + cid
    start = wid * _CQ + jnp.minimum(wid, _CR)
    n = _CQ + jnp.where(wid < _CR, 1, 0)

    @pl.when(sid == 0)
    def _():
        pltpu.sync_copy(zac_hbm, acc_sh)
        pltpu.sync_copy(zz_hbm, z_sh)

    plsc.subcore_barrier()

    sets = (
        (srcv0, dstv0, xs0, gd0, wkv0, sqb0, eab0, semidx0, semld0),
        (srcv1, dstv1, xs1, gd1, wkv1, sqb1, eab1, semidx1, semld1),
    )

    def issue_idx(s, i):
        e0 = (start + i) * C
        pltpu.async_copy(src_hbm.at[pl.ds(e0, C)], s[0], s[7])
        pltpu.async_copy(dst_hbm.at[pl.ds(e0, C)], s[1], s[7])

    def wait_idx(s):
        pltpu.make_async_copy(src_hbm.at[pl.ds(0, C)], s[0], s[7]).wait()
        pltpu.make_async_copy(dst_hbm.at[pl.ds(0, C)], s[1], s[7]).wait()

    def issue_loads(s, i):
        e0 = (start + i) * C
        pltpu.async_copy(tab_hbm.at[s[0]], s[2], s[8])
        pltpu.async_copy(tab_hbm.at[s[1]], s[3], s[8])
        pltpu.async_copy(wkv_hbm.at[pl.ds(e0, C)], s[4], s[8])
        pltpu.async_copy(sq_hbm.at[pl.ds(e0, C)], s[5], s[8])
        pltpu.async_copy(ea_hbm.at[pl.ds(e0, C)], s[6], s[8])

    def wait_loads(s):
        pltpu.make_async_copy(tab_hbm.at[s[0]], s[2], s[8]).wait()
        pltpu.make_async_copy(tab_hbm.at[s[1]], s[3], s[8]).wait()
        pltpu.make_async_copy(wkv_hbm.at[pl.ds(0, C)], s[4], s[8]).wait()
        pltpu.make_async_copy(sq_hbm.at[pl.ds(0, C)], s[5], s[8]).wait()
        pltpu.make_async_copy(ea_hbm.at[pl.ds(0, C)], s[6], s[8]).wait()

    def compute_scatter(s):
        _, dstv, xs, gd, wkvb, sqb, eab = s[:7]

        def group(j, carry2):
            b = j * 16
            dotbuf[...] = jnp.zeros((16,), jnp.float32)
            for jj in range(16):
                e = b + jj
                a16 = jnp.zeros((16,), jnp.float32)
                for k in range(4):
                    xv = plsc.bitcast(xs[e, pl.ds(16 * k, 16)], jnp.bfloat16)
                    gv = plsc.bitcast(gd[e, pl.ds(D_HALF + 16 * k, 16)],
                                      jnp.bfloat16)
                    wv_ = plsc.bitcast(wkvb[e, pl.ds(16 * k, 16)],
                                       jnp.bfloat16)
                    lo, hi = plsc.unpack(xv * gv * wv_,
                                         format=plsc.PackFormat.INTERLEAVED)
                    a16 = a16 + lo + hi
                plsc.addupdate_scatter(
                    dotbuf, [jnp.full((16,), jj, jnp.int32)], a16)
            dot16 = dotbuf[...]
            sq16 = sqb[pl.ds(b, 16)]
            ea16 = eab[pl.ds(b, 16)]
            q = jnp.exp(dot16 * ea16 * (1.0 / 256.0))
            eb[pl.ds(b, 16)] = sq16 * sq16 * q * q
            s16 = sq16 * q * ea16
            for jj in range(16):
                e = b + jj
                sval = s16[jj]
                for k in range(4):
                    wvv = plsc.bitcast(wkvb[e, pl.ds(D_HALF + 16 * k, 16)],
                                       jnp.bfloat16)
                    xv = plsc.bitcast(xs[e, pl.ds(16 * k, 16)], jnp.bfloat16)
                    lo, hi = plsc.unpack(wvv * xv,
                                         format=plsc.PackFormat.INTERLEAVED)
                    tb[e, pl.ds(16 * k, 16)] = sval * lo
                    tb[e, pl.ds(D_HALF + 16 * k, 16)] = sval * hi
            return carry2

        lax.fori_loop(0, NG, group, 0)
        pltpu.sync_copy(tb, acc_sh.at[dstv], add=True)
        pltpu.sync_copy(eb, z_sh.at[dstv], add=True)

    @pl.when(n > 0)
    def _():
        issue_idx(sets[0], 0)
        wait_idx(sets[0])
        issue_loads(sets[0], 0)

    @pl.when(n > 1)
    def _():
        issue_idx(sets[1], 1)

    def phase(i, cur, other):
        @pl.when(i + 1 < n)
        def _():
            wait_idx(other)
            issue_loads(other, i + 1)

        @pl.when(i < n)
        def _():
            wait_loads(cur)
            compute_scatter(cur)

        @pl.when(i + 2 < n)
        def _():
            issue_idx(cur, i + 2)

    def pair(p, carry):
        phase(2 * p, sets[0], sets[1])
        phase(2 * p + 1, sets[1], sets[0])
        return carry

    lax.fori_loop(0, PMAX, pair, 0)
    plsc.subcore_barrier()

    @pl.when(sid == 0)
    def _():
        pltpu.sync_copy(acc_sh, acc_out.at[cid])
        pltpu.sync_copy(z_sh, z_out.at[cid])


@functools.partial(
    pl.kernel,
    out_type=[
        jax.ShapeDtypeStruct((NC, N_NODES, D_FEAT), jnp.float32),
        jax.ShapeDtypeStruct((NC, N_NODES), jnp.float32),
    ],
    mesh=plsc.VectorSubcoreMesh(core_axis_name="c", subcore_axis_name="s"),
    compiler_params=pltpu.CompilerParams(needs_layout_passes=False),
    scratch_types=[
        pltpu.VMEM((C,), jnp.int32),
        pltpu.VMEM((C,), jnp.int32),
        pltpu.VMEM((C,), jnp.int32),
        pltpu.VMEM((C,), jnp.int32),
        pltpu.VMEM((C, D_FEAT), jnp.float32),
        pltpu.VMEM((C, D_FEAT), jnp.float32),
        pltpu.VMEM((C, D_FEAT), jnp.float32),
        pltpu.VMEM((C,), jnp.float32),
        pltpu.VMEM((C,), jnp.float32),
        pltpu.VMEM((C, D_FEAT), jnp.float32),
        pltpu.VMEM((C, D_FEAT), jnp.float32),
        pltpu.VMEM((C, D_FEAT), jnp.float32),
        pltpu.VMEM((C,), jnp.float32),
        pltpu.VMEM((C,), jnp.float32),
        pltpu.VMEM((C, D_FEAT), jnp.float32),
        pltpu.VMEM((C,), jnp.float32),
        pltpu.VMEM((16,), jnp.float32),
        pltpu.VMEM_SHARED((N_NODES, D_FEAT), jnp.float32),
        pltpu.VMEM_SHARED((N_NODES,), jnp.float32),
        pltpu.SemaphoreType.DMA,
        pltpu.SemaphoreType.DMA,
        pltpu.SemaphoreType.DMA,
        pltpu.SemaphoreType.DMA,
    ],
)
def _sc_edge_pass(*args):
    _sc_body(*args)


def _final_body(acc_ref, z_ref, wlin_ref, out_ref):
    a = acc_ref[0] + acc_ref[1]
    z = z_ref[0, :] + z_ref[1, :]
    z = jnp.where(z == 0.0, 1.0, z)
    a = a * lax.rsqrt(z)[:, None]
    out_ref[...] = jnp.dot(a, wlin_ref[...],
                           preferred_element_type=jnp.float32) * (
                               1.0 / jnp.sqrt(128.0))


def _combine(acc2, z2, W_lin):
    return pl.pallas_call(
        _final_body,
        out_shape=jax.ShapeDtypeStruct((N_NODES, D_FEAT), jnp.float32),
    )(acc2, z2, W_lin)


def kernel(edge_src, edge_dst, edge_scalar_attr, edge_attr, edge_weight_cutoff,
           node_f, W_k1, W_k2, W_dot, W_v1, W_v2, W_lin):
    ea = edge_attr.reshape(N_EDGES)
    tab = _node_prep(node_f, W_dot)
    wkv, sq = _compute_mlps(edge_scalar_attr, edge_weight_cutoff,
                            W_k1, W_k2, W_v1, W_v2)
    zac = jnp.zeros((N_NODES, D_FEAT), jnp.float32)
    zz = jnp.zeros((N_NODES,), jnp.float32)
    acc2, z2 = _sc_edge_pass(tab, wkv, sq, ea,
                             edge_src, edge_dst, zac, zz)
    return _combine(acc2, z2, W_lin)


# speedup vs baseline: 1.4567x; 1.4567x over previous
import functools

import jax
import jax.numpy as jnp
from jax import lax
from jax.experimental import pallas as pl
from jax.experimental.pallas import tpu as pltpu
from jax.experimental.pallas import tpu_sc as plsc

N_NODES = 10000
N_EDGES = 320000
D_FEAT = 128
D_HALF = 64
D_SCALAR = 16
D_HIDDEN = 64

NC = 2
NS = 16
NW = NC * NS
C = 32
NCHUNK = N_EDGES // C
_CQ, _CR = divmod(NCHUNK, NW)
PMAX = (_CQ + 1 + 1) // 2
NG = C // 16


def _pack_pairs(w):
    lo = lax.bitcast_convert_type(w[:, :D_HALF].astype(jnp.bfloat16), jnp.uint16)
    hi = lax.bitcast_convert_type(w[:, D_HALF:].astype(jnp.bfloat16), jnp.uint16)
    word = lo.astype(jnp.uint32) | (hi.astype(jnp.uint32) << 16)
    return lax.bitcast_convert_type(word, jnp.float32)


def _g_body(nf_ref, wdot_ref, tab_ref):
    nf = nf_ref[...]
    g = jnp.dot(nf, wdot_ref[...], preferred_element_type=jnp.float32)
    tab_ref[:, :D_HALF] = _pack_pairs(nf)
    tab_ref[:, D_HALF:] = _pack_pairs(g)


def _node_prep(node_f, W_dot):
    return pl.pallas_call(
        _g_body,
        out_shape=jax.ShapeDtypeStruct((N_NODES, D_FEAT), jnp.float32),
    )(node_f, W_dot)


_BE = 8000


def _mlp_body(esc_ref, cut_ref, wk1_ref, wk2_ref, wv1_ref, wv2_ref,
              wkv_ref, sq_ref):
    esc = esc_ref[...]
    hk = jnp.maximum(jnp.dot(esc, wk1_ref[...],
                             preferred_element_type=jnp.float32), 0.0)
    wk = jnp.dot(hk, wk2_ref[...],
                 preferred_element_type=jnp.float32) * (1.0 / 32.0)
    hv = jnp.maximum(jnp.dot(esc, wv1_ref[...],
                             preferred_element_type=jnp.float32), 0.0)
    wv = jnp.dot(hv, wv2_ref[...],
                 preferred_element_type=jnp.float32) * (1.0 / 32.0)
    wkv_ref[:, :D_HALF] = _pack_pairs(wk)
    wkv_ref[:, D_HALF:] = _pack_pairs(wv)
    sq_ref[...] = jnp.sqrt(cut_ref[...])


def _compute_mlps(esc, cutoff, W_k1, W_k2, W_v1, W_v2):
    nb = N_EDGES // _BE
    cut3 = cutoff.reshape(nb, 1, _BE)
    wkv, sq3 = pl.pallas_call(
        _mlp_body,
        grid=(nb,),
        in_specs=[
            pl.BlockSpec((_BE, D_SCALAR), lambda i: (i, 0)),
            pl.BlockSpec((1, 1, _BE), lambda i: (i, 0, 0)),
            pl.BlockSpec((D_SCALAR, D_HIDDEN), lambda i: (0, 0)),
            pl.BlockSpec((D_HIDDEN, D_FEAT), lambda i: (0, 0)),
            pl.BlockSpec((D_SCALAR, D_HIDDEN), lambda i: (0, 0)),
            pl.BlockSpec((D_HIDDEN, D_FEAT), lambda i: (0, 0)),
        ],
        out_specs=[
            pl.BlockSpec((_BE, D_FEAT), lambda i: (i, 0)),
            pl.BlockSpec((1, 1, _BE), lambda i: (i, 0, 0)),
        ],
        out_shape=[
            jax.ShapeDtypeStruct((N_EDGES, D_FEAT), jnp.float32),
            jax.ShapeDtypeStruct((nb, 1, _BE), jnp.float32),
        ],
    )(esc, cut3, W_k1, W_k2, W_v1, W_v2)
    return wkv, sq3.reshape(N_EDGES)


def _sc_body(tab_hbm, wkv_hbm, sq_hbm, ea_hbm, src_hbm, dst_hbm,
             zac_hbm, zz_hbm,
             acc_out, z_out,
             srcv0, dstv0, srcv1, dstv1,
             xs0, gd0, wkv0, sqb0, eab0,
             xs1, gd1, wkv1, sqb1, eab1,
             tb0, eb0, dsc0, tb1, eb1, dsc1,
             acc_sh, z_sh, semidx0, semidx1, semld0, semld1,
             semdsc0, semdsc1, semsc0, semsc1):
    cid = lax.axis_index("c")
    sid = lax.axis_index("s")
    wid = sid * NC + cid
    start = wid * _CQ + jnp.minimum(wid, _CR)
    n = _CQ + jnp.where(wid < _CR, 1, 0)

    @pl.when(sid == 0)
    def _():
        pltpu.sync_copy(zac_hbm, acc_sh)
        pltpu.sync_copy(zz_hbm, z_sh)

    plsc.subcore_barrier()

    sets = (
        (srcv0, dstv0, xs0, gd0, wkv0, sqb0, eab0, semidx0, semld0,
         tb0, eb0, dsc0, semdsc0, semsc0),
        (srcv1, dstv1, xs1, gd1, wkv1, sqb1, eab1, semidx1, semld1,
         tb1, eb1, dsc1, semdsc1, semsc1),
    )

    def issue_idx(s, i):
        e0 = (start + i) * C
        pltpu.async_copy(src_hbm.at[pl.ds(e0, C)], s[0], s[7])
        pltpu.async_copy(dst_hbm.at[pl.ds(e0, C)], s[1], s[7])

    def wait_idx(s):
        pltpu.make_async_copy(src_hbm.at[pl.ds(0, C)], s[0], s[7]).wait()
        pltpu.make_async_copy(dst_hbm.at[pl.ds(0, C)], s[1], s[7]).wait()

    def issue_loads(s, i):
        e0 = (start + i) * C
        pltpu.async_copy(tab_hbm.at[s[0]], s[2], s[8])
        pltpu.async_copy(tab_hbm.at[s[1]], s[3], s[8])
        pltpu.async_copy(wkv_hbm.at[pl.ds(e0, C)], s[4], s[8])
        pltpu.async_copy(sq_hbm.at[pl.ds(e0, C)], s[5], s[8])
        pltpu.async_copy(ea_hbm.at[pl.ds(e0, C)], s[6], s[8])

    def wait_loads(s):
        pltpu.make_async_copy(tab_hbm.at[s[0]], s[2], s[8]).wait()
        pltpu.make_async_copy(tab_hbm.at[s[1]], s[3], s[8]).wait()
        pltpu.make_async_copy(wkv_hbm.at[pl.ds(0, C)], s[4], s[8]).wait()
        pltpu.make_async_copy(sq_hbm.at[pl.ds(0, C)], s[5], s[8]).wait()
        pltpu.make_async_copy(ea_hbm.at[pl.ds(0, C)], s[6], s[8]).wait()

    def issue_dsc(s, i):
        e0 = (start + i) * C
        pltpu.async_copy(dst_hbm.at[pl.ds(e0, C)], s[11], s[12])

    def wait_dsc(s):
        pltpu.make_async_copy(dst_hbm.at[pl.ds(0, C)], s[11], s[12]).wait()

    def issue_scatter(s):
        pltpu.async_copy(s[9], acc_sh.at[s[11]], s[13], add=True)
        pltpu.async_copy(s[10], z_sh.at[s[11]], s[13], add=True)

    def wait_scatter(s):
        pltpu.make_async_copy(s[9], acc_sh.at[s[11]], s[13]).wait()
        pltpu.make_async_copy(s[10], z_sh.at[s[11]], s[13]).wait()

    def compute_scatter(s):
        _, dstv, xs, gd, wkvb, sqb, eab = s[:7]
        tb, eb = s[9], s[10]

        def group(j, carry2):
            b = j * 16
            lane = lax.iota(jnp.int32, 16)
            dot16 = jnp.zeros((16,), jnp.float32)
            for jj in range(16):
                e = b + jj
                a16 = jnp.zeros((16,), jnp.float32)
                for k in range(4):
                    xv = plsc.bitcast(xs[e, pl.ds(16 * k, 16)], jnp.bfloat16)
                    gv = plsc.bitcast(gd[e, pl.ds(D_HALF + 16 * k, 16)],
                                      jnp.bfloat16)
                    wv_ = plsc.bitcast(wkvb[e, pl.ds(16 * k, 16)],
                                       jnp.bfloat16)
                    lo, hi = plsc.unpack(xv * gv * wv_,
                                         format=plsc.PackFormat.INTERLEAVED)
                    a16 = a16 + lo + hi
                dot16 = jnp.where(lane == jj, jnp.sum(a16), dot16)
            sq16 = sqb[pl.ds(b, 16)]
            ea16 = eab[pl.ds(b, 16)]
            q = jnp.exp(dot16 * ea16 * (1.0 / 256.0))
            eb[pl.ds(b, 16)] = sq16 * sq16 * q * q
            s16 = sq16 * q * ea16
            for jj in range(16):
                e = b + jj
                sval = s16[jj]
                for k in range(4):
                    wvv = plsc.bitcast(wkvb[e, pl.ds(D_HALF + 16 * k, 16)],
                                       jnp.bfloat16)
                    xv = plsc.bitcast(xs[e, pl.ds(16 * k, 16)], jnp.bfloat16)
                    lo, hi = plsc.unpack(wvv * xv,
                                         format=plsc.PackFormat.INTERLEAVED)
                    tb[e, pl.ds(16 * k, 16)] = sval * lo
                    tb[e, pl.ds(D_HALF + 16 * k, 16)] = sval * hi
            return carry2

        lax.fori_loop(0, NG, group, 0)

    @pl.when(n > 0)
    def _():
        issue_idx(sets[0], 0)
        wait_idx(sets[0])
        issue_loads(sets[0], 0)

    @pl.when(n > 1)
    def _():
        issue_idx(sets[1], 1)

    def phase(i, cur, other):
        @pl.when(i + 1 < n)
        def _():
            wait_idx(other)
            issue_loads(other, i + 1)

        @pl.when(i < n)
        def _():
            @pl.when(i >= 2)
            def _():
                wait_scatter(cur)
            issue_dsc(cur, i)
            wait_loads(cur)
            compute_scatter(cur)
            wait_dsc(cur)
            issue_scatter(cur)

        @pl.when(i + 2 < n)
        def _():
            issue_idx(cur, i + 2)

    def pair(p, carry):
        phase(2 * p, sets[0], sets[1])
        phase(2 * p + 1, sets[1], sets[0])
        return carry

    lax.fori_loop(0, PMAX, pair, 0)
    wait_scatter(sets[0])
    wait_scatter(sets[1])
    plsc.subcore_barrier()

    @pl.when(sid == 0)
    def _():
        pltpu.sync_copy(acc_sh, acc_out.at[cid])
        pltpu.sync_copy(z_sh, z_out.at[cid])


@functools.partial(
    pl.kernel,
    out_type=[
        jax.ShapeDtypeStruct((NC, N_NODES, D_FEAT), jnp.float32),
        jax.ShapeDtypeStruct((NC, N_NODES), jnp.float32),
    ],
    mesh=plsc.VectorSubcoreMesh(core_axis_name="c", subcore_axis_name="s"),
    compiler_params=pltpu.CompilerParams(needs_layout_passes=False),
    scratch_types=[
        pltpu.VMEM((C,), jnp.int32),
        pltpu.VMEM((C,), jnp.int32),
        pltpu.VMEM((C,), jnp.int32),
        pltpu.VMEM((C,), jnp.int32),
        pltpu.VMEM((C, D_FEAT), jnp.float32),
        pltpu.VMEM((C, D_FEAT), jnp.float32),
        pltpu.VMEM((C, D_FEAT), jnp.float32),
        pltpu.VMEM((C,), jnp.float32),
        pltpu.VMEM((C,), jnp.float32),
        pltpu.VMEM((C, D_FEAT), jnp.float32),
        pltpu.VMEM((C, D_FEAT), jnp.float32),
        pltpu.VMEM((C, D_FEAT), jnp.float32),
        pltpu.VMEM((C,), jnp.float32),
        pltpu.VMEM((C,), jnp.float32),
        pltpu.VMEM((C, D_FEAT), jnp.float32),
        pltpu.VMEM((C,), jnp.float32),
        pltpu.VMEM((C,), jnp.int32),
        pltpu.VMEM((C, D_FEAT), jnp.float32),
        pltpu.VMEM((C,), jnp.float32),
        pltpu.VMEM((C,), jnp.int32),
        pltpu.VMEM_SHARED((N_NODES, D_FEAT), jnp.float32),
        pltpu.VMEM_SHARED((N_NODES,), jnp.float32),
        pltpu.SemaphoreType.DMA,
        pltpu.SemaphoreType.DMA,
        pltpu.SemaphoreType.DMA,
        pltpu.SemaphoreType.DMA,
        pltpu.SemaphoreType.DMA,
        pltpu.SemaphoreType.DMA,
        pltpu.SemaphoreType.DMA,
        pltpu.SemaphoreType.DMA,
    ],
)
def _sc_edge_pass(*args):
    _sc_body(*args)


def _final_body(acc_ref, z_ref, wlin_ref, out_ref):
    a = acc_ref[0] + acc_ref[1]
    z = z_ref[0, :] + z_ref[1, :]
    z = jnp.where(z == 0.0, 1.0, z)
    a = a * lax.rsqrt(z)[:, None]
    out_ref[...] = jnp.dot(a, wlin_ref[...],
                           preferred_element_type=jnp.float32) * (
                               1.0 / jnp.sqrt(128.0))


def _combine(acc2, z2, W_lin):
    return pl.pallas_call(
        _final_body,
        out_shape=jax.ShapeDtypeStruct((N_NODES, D_FEAT), jnp.float32),
    )(acc2, z2, W_lin)


def kernel(edge_src, edge_dst, edge_scalar_attr, edge_attr, edge_weight_cutoff,
           node_f, W_k1, W_k2, W_dot, W_v1, W_v2, W_lin):
    ea = edge_attr.reshape(N_EDGES)
    tab = _node_prep(node_f, W_dot)
    wkv, sq = _compute_mlps(edge_scalar_attr, edge_weight_cutoff,
                            W_k1, W_k2, W_v1, W_v2)
    zac = jnp.zeros((N_NODES, D_FEAT), jnp.float32)
    zz = jnp.zeros((N_NODES,), jnp.float32)
    acc2, z2 = _sc_edge_pass(tab, wkv, sq, ea,
                             edge_src, edge_dst, zac, zz)
    return _combine(acc2, z2, W_lin)
